# baseline (speedup 1.0000x reference)
import jax
import jax.numpy as jnp
from jax import lax
from jax.experimental import pallas as pl
from jax.experimental.pallas import tpu as pltpu
from jax.experimental.pallas import tpu_sc as plsc

_B = 16384
_D = 16
_NW = 32
_BPW = _B // _NW
_G = 16
_OW = 4 * _D


def _body(uid_hbm, cid_hbm, wd_hbm, tf_hbm,
          ut_hbm, ct_hbm, wt_hbm, tt_hbm,
          out_hbm,
          iu, ic, iw, it, cbuf, wbuf, tbuf, out_v, sem, gsem):
    wid = lax.axis_index("s") * 2 + lax.axis_index("c")
    base = wid * _BPW

    pltpu.sync_copy(uid_hbm.at[pl.ds(base, _BPW)], iu)
    pltpu.sync_copy(cid_hbm.at[pl.ds(base, _BPW)], ic)
    pltpu.sync_copy(wd_hbm.at[pl.ds(base, _BPW)], iw)
    pltpu.sync_copy(tf_hbm.at[pl.ds(base, _BPW)], it)
    cp_c = pltpu.async_copy(ct_hbm, cbuf, sem)
    cp_w = pltpu.async_copy(wt_hbm, wbuf, sem)
    cp_t = pltpu.async_copy(tt_hbm, tbuf, sem)
    cp_c.wait()
    cp_w.wait()
    cp_t.wait()

    def step(g, _):
        uo = iu[pl.ds(g * _G, _G)] * _D
        co = ic[pl.ds(g * _G, _G)] * _D
        wo = iw[pl.ds(g * _G, _G)] * _D
        to = it[pl.ds(g * _G, _G)] * _D
        for j in range(_G):
            slab = pl.multiple_of((g * _G + j) * _OW, _OW)
            pltpu.async_copy(ut_hbm.at[pl.ds(pl.multiple_of(uo[j], _D), _D)],
                             out_v.at[pl.ds(slab, _D)], gsem)
            out_v[pl.ds(slab + _D, _D)] = cbuf[pl.ds(pl.multiple_of(co[j], _D), _D)]
            out_v[pl.ds(slab + 2 * _D, _D)] = wbuf[pl.ds(pl.multiple_of(wo[j], _D), _D)]
            out_v[pl.ds(slab + 3 * _D, _D)] = tbuf[pl.ds(pl.multiple_of(to[j], _D), _D)]
        return 0

    lax.fori_loop(0, _BPW // _G, step, 0)

    pltpu.make_async_copy(
        ut_hbm.at[pl.ds(0, _BPW * _D)], cbuf.at[pl.ds(0, _BPW * _D)], gsem
    ).wait()

    pltpu.sync_copy(out_v, out_hbm.at[pl.ds(base * _OW, _BPW * _OW)])


@jax.jit
def _run(uid, cid, wd, tf, ut, ct, wt, tt):
    mesh = plsc.VectorSubcoreMesh(core_axis_name="c", subcore_axis_name="s")
    out_flat = pl.kernel(
        _body,
        out_type=jax.ShapeDtypeStruct((_B * _OW,), jnp.float32),
        mesh=mesh,
        scratch_types=[
            pltpu.VMEM((_BPW,), jnp.int32),
            pltpu.VMEM((_BPW,), jnp.int32),
            pltpu.VMEM((_BPW,), jnp.int32),
            pltpu.VMEM((_BPW,), jnp.int32),
            pltpu.VMEM((1001 * _D,), jnp.float32),
            pltpu.VMEM((8 * _D,), jnp.float32),
            pltpu.VMEM((25 * _D,), jnp.float32),
            pltpu.VMEM((_BPW * _OW,), jnp.float32),
            pltpu.SemaphoreType.DMA,
            pltpu.SemaphoreType.DMA,
        ],
    )(uid, cid, wd, tf,
      ut.reshape(-1), ct.reshape(-1), wt.reshape(-1), tt.reshape(-1))
    return out_flat.reshape(_B, _OW)


def kernel(user_id, category_id, weekday, time_frames,
           user_table, category_table, weekday_table, time_frame_table):
    return _run(user_id.astype(jnp.int32), category_id.astype(jnp.int32),
                weekday.astype(jnp.int32), time_frames.astype(jnp.int32),
                user_table, category_table, weekday_table, time_frame_table)


# speedup vs baseline: 5.1901x; 5.1901x over previous
import jax
import jax.numpy as jnp
from jax import lax
from jax.experimental import pallas as pl
from jax.experimental.pallas import tpu as pltpu
from jax.experimental.pallas import tpu_sc as plsc

_B = 16384
_D = 16
_NW = 32
_BPW = _B // _NW
_G = 16
_NWAVE = _BPW // _G


def _body(uid_hbm, cid_hbm, wd_hbm, tf_hbm,
          ut_hbm, ct_hbm, wt_hbm, tt_hbm,
          out_hbm,
          iu, ic, iw, it, cbuf, wbuf, tbuf, chunk, slab, sem, g0sem, g1sem):
    wid = lax.axis_index("s") * 2 + lax.axis_index("c")
    base = wid * _BPW
    iota = lax.iota(jnp.int32, _D)

    pltpu.sync_copy(uid_hbm.at[pl.ds(base, _BPW)], iu)
    pltpu.sync_copy(cid_hbm.at[pl.ds(base, _BPW)], ic)
    pltpu.sync_copy(wd_hbm.at[pl.ds(base, _BPW)], iw)
    pltpu.sync_copy(tf_hbm.at[pl.ds(base, _BPW)], it)
    cp_c = pltpu.async_copy(ct_hbm, cbuf, sem)
    cp_w = pltpu.async_copy(wt_hbm, wbuf, sem)
    cp_t = pltpu.async_copy(tt_hbm, tbuf, sem)
    cp_c.wait()
    cp_w.wait()
    cp_t.wait()

    gsems = (g0sem, g1sem)

    def fire(g, half):
        uvec = iu[pl.ds(g * _G, _G)]
        starts = uvec & jnp.int32(-128)
        for j in range(_G):
            pltpu.async_copy(
                ut_hbm.at[:, pl.ds(pl.multiple_of(starts[j], 128), 128)],
                chunk.at[half * _G + j],
                gsems[half],
            )

    def drain(half):
        for j in range(_G):
            pltpu.make_async_copy(
                ut_hbm.at[:, pl.ds(0, 128)], chunk.at[half * _G + j],
                gsems[half],
            ).wait()

    def process(g, half):
        uvec = iu[pl.ds(g * _G, _G)]
        lanes = uvec & jnp.int32(127)
        cvec = ic[pl.ds(g * _G, _G)]
        wvec = iw[pl.ds(g * _G, _G)]
        tvec = it[pl.ds(g * _G, _G)]
        for j in range(_G):
            col = jnp.broadcast_to(g * _G + j, (_D,)).astype(jnp.int32)
            u16 = plsc.load_gather(
                chunk.at[half * _G + j],
                [iota, jnp.broadcast_to(lanes[j], (_D,))],
            )
            c16 = plsc.load_gather(
                cbuf, [iota, jnp.broadcast_to(cvec[j], (_D,))])
            w16 = plsc.load_gather(
                wbuf, [iota, jnp.broadcast_to(wvec[j], (_D,))])
            t16 = plsc.load_gather(
                tbuf, [iota, jnp.broadcast_to(tvec[j], (_D,))])
            plsc.store_scatter(slab, [iota, col], u16)
            plsc.store_scatter(slab, [iota + _D, col], c16)
            plsc.store_scatter(slab, [iota + 2 * _D, col], w16)
            plsc.store_scatter(slab, [iota + 3 * _D, col], t16)

    fire(0, 0)

    def pair(p, _):
        g0 = 2 * p
        fire(g0 + 1, 1)
        drain(0)
        process(g0, 0)

        @pl.when(p < _NWAVE // 2 - 1)
        def _():
            fire(g0 + 2, 0)

        drain(1)
        process(g0 + 1, 1)
        return 0

    lax.fori_loop(0, _NWAVE // 2, pair, 0)

    pltpu.sync_copy(slab, out_hbm.at[:, pl.ds(base, _BPW)])


@jax.jit
def _run(uid, cid, wd, tf, utT, ctT, wtT, ttT):
    mesh = plsc.VectorSubcoreMesh(core_axis_name="c", subcore_axis_name="s")
    out_t = pl.kernel(
        _body,
        out_type=jax.ShapeDtypeStruct((4 * _D, _B), jnp.float32),
        mesh=mesh,
        scratch_types=[
            pltpu.VMEM((_BPW,), jnp.int32),
            pltpu.VMEM((_BPW,), jnp.int32),
            pltpu.VMEM((_BPW,), jnp.int32),
            pltpu.VMEM((_BPW,), jnp.int32),
            pltpu.VMEM((_D, 1001), jnp.float32),
            pltpu.VMEM((_D, 8), jnp.float32),
            pltpu.VMEM((_D, 25), jnp.float32),
            pltpu.VMEM((2 * _G, _D, 128), jnp.float32),
            pltpu.VMEM((4 * _D, _BPW), jnp.float32),
            pltpu.SemaphoreType.DMA,
            pltpu.SemaphoreType.DMA,
            pltpu.SemaphoreType.DMA,
        ],
        compiler_params=pltpu.CompilerParams(needs_layout_passes=False),
    )(uid, cid, wd, tf, utT, ctT, wtT, ttT)
    return out_t.T


def kernel(user_id, category_id, weekday, time_frames,
           user_table, category_table, weekday_table, time_frame_table):
    return _run(user_id.astype(jnp.int32), category_id.astype(jnp.int32),
                weekday.astype(jnp.int32), time_frames.astype(jnp.int32),
                user_table.T, category_table.T, weekday_table.T,
                time_frame_table.T)


# speedup vs baseline: 5.6489x; 1.0884x over previous
import jax
import jax.numpy as jnp
from jax import lax
from jax.experimental import pallas as pl
from jax.experimental.pallas import tpu as pltpu
from jax.experimental.pallas import tpu_sc as plsc

_B = 16384
_D = 16
_NW = 32
_BPW = _B // _NW
_G = 16
_NWAVE = _BPW // _G


def _body(uid_hbm, cid_hbm, wd_hbm, tf_hbm,
          ut_hbm, ct_hbm, wt_hbm, tt_hbm,
          out_hbm,
          iu, ic, iw, it, cbuf, wbuf, tbuf, chunk, slab, sem, g0sem, g1sem):
    wid = lax.axis_index("s") * 2 + lax.axis_index("c")
    base = wid * _BPW
    iota = lax.iota(jnp.int32, _D)

    pltpu.sync_copy(uid_hbm.at[pl.ds(base, _BPW)], iu)
    pltpu.sync_copy(cid_hbm.at[pl.ds(base, _BPW)], ic)
    pltpu.sync_copy(wd_hbm.at[pl.ds(base, _BPW)], iw)
    pltpu.sync_copy(tf_hbm.at[pl.ds(base, _BPW)], it)
    cp_c = pltpu.async_copy(ct_hbm, cbuf, sem)
    cp_w = pltpu.async_copy(wt_hbm, wbuf, sem)
    cp_t = pltpu.async_copy(tt_hbm, tbuf, sem)
    cp_c.wait()
    cp_w.wait()
    cp_t.wait()

    gsems = (g0sem, g1sem)

    def fire(g, half):
        uvec = iu[pl.ds(g * _G, _G)]
        starts = uvec & jnp.int32(-128)
        for j in range(_G):
            pltpu.async_copy(
                ut_hbm.at[:, pl.ds(pl.multiple_of(starts[j], 128), 128)],
                chunk.at[half * _G + j],
                gsems[half],
            )

    def drain(half):
        for j in range(_G):
            pltpu.make_async_copy(
                ut_hbm.at[:, pl.ds(0, 128)], chunk.at[half * _G + j],
                gsems[half],
            ).wait()

    def process(g, half):
        uvec = iu[pl.ds(g * _G, _G)]
        lanes = uvec & jnp.int32(127)
        cvec = ic[pl.ds(g * _G, _G)]
        wvec = iw[pl.ds(g * _G, _G)]
        tvec = it[pl.ds(g * _G, _G)]
        wave = chunk.at[pl.ds(half * _G, _G)]
        col = pl.multiple_of(g * _G, _G)
        for c in range(_D):
            cc = jnp.broadcast_to(jnp.int32(c), (_D,))
            slab[c, pl.ds(col, _G)] = plsc.load_gather(
                wave, [iota, cc, lanes])
            slab[_D + c, pl.ds(col, _G)] = plsc.load_gather(cbuf, [cc, cvec])
            slab[2 * _D + c, pl.ds(col, _G)] = plsc.load_gather(
                wbuf, [cc, wvec])
            slab[3 * _D + c, pl.ds(col, _G)] = plsc.load_gather(
                tbuf, [cc, tvec])

    fire(0, 0)

    def pair(p, _):
        g0 = 2 * p
        fire(g0 + 1, 1)
        drain(0)
        process(g0, 0)

        @pl.when(p < _NWAVE // 2 - 1)
        def _():
            fire(g0 + 2, 0)

        drain(1)
        process(g0 + 1, 1)
        return 0

    lax.fori_loop(0, _NWAVE // 2, pair, 0)

    pltpu.sync_copy(slab, out_hbm.at[:, pl.ds(base, _BPW)])


@jax.jit
def _run(uid, cid, wd, tf, utT, ctT, wtT, ttT):
    mesh = plsc.VectorSubcoreMesh(core_axis_name="c", subcore_axis_name="s")
    out_t = pl.kernel(
        _body,
        out_type=jax.ShapeDtypeStruct((4 * _D, _B), jnp.float32),
        mesh=mesh,
        scratch_types=[
            pltpu.VMEM((_BPW,), jnp.int32),
            pltpu.VMEM((_BPW,), jnp.int32),
            pltpu.VMEM((_BPW,), jnp.int32),
            pltpu.VMEM((_BPW,), jnp.int32),
            pltpu.VMEM((_D, 1001), jnp.float32),
            pltpu.VMEM((_D, 8), jnp.float32),
            pltpu.VMEM((_D, 25), jnp.float32),
            pltpu.VMEM((2 * _G, _D, 128), jnp.float32),
            pltpu.VMEM((4 * _D, _BPW), jnp.float32),
            pltpu.SemaphoreType.DMA,
            pltpu.SemaphoreType.DMA,
            pltpu.SemaphoreType.DMA,
        ],
        compiler_params=pltpu.CompilerParams(needs_layout_passes=False),
    )(uid, cid, wd, tf, utT, ctT, wtT, ttT)
    return out_t.T


def kernel(user_id, category_id, weekday, time_frames,
           user_table, category_table, weekday_table, time_frame_table):
    return _run(user_id.astype(jnp.int32), category_id.astype(jnp.int32),
                weekday.astype(jnp.int32), time_frames.astype(jnp.int32),
                user_table.T, category_table.T, weekday_table.T,
                time_frame_table.T)
